# baseline (speedup 1.0000x reference)
import jax
import jax.numpy as jnp
from jax.experimental import pallas as pl
from jax.experimental.pallas import tpu as pltpu

_BN = 1024


def _assign_kernel(x_ref, c_ref, out_ref, c_sq_ref):
    x = x_ref[...]
    c = c_ref[...]

    @pl.when(pl.program_id(0) == 0)
    def _():
        c_sq_ref[...] = jnp.sum(c * c, axis=0, keepdims=True)

    x_sq = jnp.sum(x * x, axis=1, keepdims=True)
    cross = jax.lax.dot_general(
        x, c, (((1,), (0,)), ((), ())),
        preferred_element_type=jnp.float32)
    scores = x_sq - 2.0 * cross + c_sq_ref[...]
    out_ref[...] = jnp.argmin(scores, axis=1).astype(jnp.int32)


def kernel(test_features, centroids):
    n, d = test_features.shape
    k = centroids.shape[1]
    return pl.pallas_call(
        _assign_kernel,
        grid=(n // _BN,),
        in_specs=[
            pl.BlockSpec((_BN, d), lambda i: (i, 0)),
            pl.BlockSpec((d, k), lambda i: (0, 0)),
        ],
        out_specs=pl.BlockSpec((_BN,), lambda i: (i,)),
        out_shape=jax.ShapeDtypeStruct((n,), jnp.int32),
        scratch_shapes=[pltpu.VMEM((1, k), jnp.float32)],
    )(test_features, centroids)


# speedup vs baseline: 1.0267x; 1.0267x over previous
import jax
import jax.numpy as jnp
from jax.experimental import pallas as pl

_BN = 1024


def _c_sq_kernel(c_ref, c_sq_ref):
    c = c_ref[...]
    c_sq_ref[...] = jnp.sum(c * c, axis=0, keepdims=True)


def _assign_kernel(x_ref, c_ref, c_sq_ref, out_ref):
    x = x_ref[...]
    c = c_ref[...]
    x_sq = jnp.sum(x * x, axis=1, keepdims=True)
    cross = jax.lax.dot_general(
        x, c, (((1,), (0,)), ((), ())),
        preferred_element_type=jnp.float32)
    scores = x_sq - 2.0 * cross + c_sq_ref[...]
    out_ref[...] = jnp.argmin(scores, axis=1).astype(jnp.int32)


def kernel(test_features, centroids):
    n, d = test_features.shape
    k = centroids.shape[1]
    c_sq = pl.pallas_call(
        _c_sq_kernel,
        out_shape=jax.ShapeDtypeStruct((1, k), jnp.float32),
    )(centroids)
    return pl.pallas_call(
        _assign_kernel,
        grid=(n // _BN,),
        in_specs=[
            pl.BlockSpec((_BN, d), lambda i: (i, 0)),
            pl.BlockSpec((d, k), lambda i: (0, 0)),
            pl.BlockSpec((1, k), lambda i: (0, 0)),
        ],
        out_specs=pl.BlockSpec((_BN,), lambda i: (i,)),
        out_shape=jax.ShapeDtypeStruct((n,), jnp.int32),
    )(test_features, centroids, c_sq)


# speedup vs baseline: 1.1188x; 1.0896x over previous
import jax
import jax.numpy as jnp
from jax.experimental import pallas as pl

_BN = 512


def _assign_kernel(x_ref, c_ref, out_ref):
    x = x_ref[...]
    c = c_ref[...]
    x_sq = jnp.sum(x * x, axis=1, keepdims=True)
    c_sq = jnp.sum(c * c, axis=0, keepdims=True)
    cross = jax.lax.dot_general(
        x, c, (((1,), (0,)), ((), ())),
        preferred_element_type=jnp.float32)
    scores = x_sq - 2.0 * cross + c_sq
    out_ref[...] = jnp.argmin(scores, axis=1).astype(jnp.int32)


def kernel(test_features, centroids):
    n, d = test_features.shape
    k = centroids.shape[1]
    return pl.pallas_call(
        _assign_kernel,
        grid=(n // _BN,),
        in_specs=[
            pl.BlockSpec((_BN, d), lambda i: (i, 0)),
            pl.BlockSpec((d, k), lambda i: (0, 0)),
        ],
        out_specs=pl.BlockSpec((_BN,), lambda i: (i,)),
        out_shape=jax.ShapeDtypeStruct((n,), jnp.int32),
    )(test_features, centroids)
